# baseline (speedup 1.0000x reference)
import functools

import jax
import jax.numpy as jnp
from jax import lax
from jax.experimental import pallas as pl
from jax.experimental.pallas import tpu as pltpu
from jax.experimental.pallas import tpu_sc as plsc

N_ITEMS = 1000000
H = 64
BATCH = 16384
BN = 32768

NC = 2
NS = 16
L = 16
NW = NC * NS
BPW = BATCH // NW
CHUNK = 128
NCHUNK = BPW // CHUNK

_mesh = plsc.VectorSubcoreMesh(core_axis_name="c", subcore_axis_name="s")


def _matvec_body(w_ref, t_ref, o_ref):
    res = jnp.dot(w_ref[...], t_ref[...], preferred_element_type=jnp.float32)
    o_ref[...] = res[0]


def _matvec(lin_w, table_t):
    return pl.pallas_call(
        _matvec_body,
        grid=(pl.cdiv(N_ITEMS, BN),),
        in_specs=[
            pl.BlockSpec((1, H), lambda i: (0, 0)),
            pl.BlockSpec((H, BN), lambda i: (0, i)),
        ],
        out_specs=pl.BlockSpec((BN,), lambda i: (i,)),
        out_shape=jax.ShapeDtypeStruct((N_ITEMS,), jnp.float32),
    )(lin_w, table_t)


@functools.partial(
    pl.kernel,
    mesh=_mesh,
    out_type=jax.ShapeDtypeStruct((NW, NCHUNK, CHUNK), jnp.float32),
    compiler_params=pltpu.CompilerParams(
        needs_layout_passes=False, use_tc_tiling_on_sc=False
    ),
    scratch_types=[
        pltpu.VMEM((NCHUNK, CHUNK), jnp.int32),
        pltpu.VMEM((NCHUNK, CHUNK), jnp.float32),
        pltpu.VMEM((L,), jnp.float32),
        [pltpu.SemaphoreType.DMA] * NCHUNK,
    ],
)
def _gather(idx_hbm, y_hbm, b_hbm, out_hbm, idx_v, val_v, b_v, sems):
    wid = lax.axis_index("s") * NC + lax.axis_index("c")
    pltpu.sync_copy(idx_hbm.at[wid], idx_v)
    pltpu.sync_copy(b_hbm, b_v)
    copies = [
        pltpu.async_copy(y_hbm.at[idx_v.at[j]], val_v.at[j], sems[j])
        for j in range(NCHUNK)
    ]
    bias = b_v[pl.ds(0, L)][0]
    for j in range(NCHUNK):
        copies[j].wait()
        for q in range(CHUNK // L):
            val_v[j, pl.ds(q * L, L)] = val_v[j, pl.ds(q * L, L)] + bias
    pltpu.sync_copy(val_v, out_hbm.at[wid])


def kernel(users, items, emb_table, lin_w, lin_b):
    del users
    y_full = _matvec(lin_w, emb_table.T)
    idx = items.astype(jnp.int32).reshape(NW, NCHUNK, CHUNK)
    b16 = jnp.pad(lin_b.reshape(1), (0, L - 1))
    out = _gather(idx, y_full, b16)
    return out.reshape(BATCH, 1)
